# baseline (speedup 1.0000x reference)
import functools

import jax
import jax.numpy as jnp
from jax import lax
from jax.experimental import pallas as pl
from jax.experimental.pallas import tpu as pltpu
from jax.experimental.pallas import tpu_sc as plsc

VOCAB = 1000000
EMBED_DIM = 64
BATCH = 16384

_info = plsc.get_sparse_core_info()
_NC, _NS, _L = _info.num_cores, _info.num_subcores, _info.num_lanes
_NW = _NC * _NS
_BPW = BATCH // _NW
_CHUNK = 128
_NCH = _BPW // _CHUNK
_NG = _CHUNK // _L


def _glove_gather(center_idx, context_idx, center_embed, context_embed,
                  center_bias16, context_bias16):
    mesh = plsc.VectorSubcoreMesh(core_axis_name="c", subcore_axis_name="s")

    @functools.partial(
        pl.kernel,
        mesh=mesh,
        compiler_params=pltpu.CompilerParams(use_tc_tiling_on_sc=False,
                                             needs_layout_passes=False),
        out_type=[
            jax.ShapeDtypeStruct((BATCH, EMBED_DIM), jnp.float32),
            jax.ShapeDtypeStruct((BATCH, EMBED_DIM), jnp.float32),
            jax.ShapeDtypeStruct((BATCH,), jnp.float32),
            jax.ShapeDtypeStruct((BATCH,), jnp.float32),
        ],
        scratch_types=[
            pltpu.VMEM((_NCH, _CHUNK), jnp.int32),
            pltpu.VMEM((_NCH, _CHUNK), jnp.int32),
            pltpu.VMEM((_NCH, _CHUNK), jnp.int32),
            pltpu.VMEM((_NCH, _CHUNK), jnp.int32),
            pltpu.VMEM((_NCH, _CHUNK, EMBED_DIM), jnp.float32),
            pltpu.VMEM((_NCH, _CHUNK, EMBED_DIM), jnp.float32),
            pltpu.VMEM((_NCH, _CHUNK, _L), jnp.float32),
            pltpu.VMEM((_NCH, _CHUNK, _L), jnp.float32),
            pltpu.VMEM((_NCH, _CHUNK), jnp.float32),
            pltpu.VMEM((_NCH, _CHUNK), jnp.float32),
            pltpu.SemaphoreType.DMA,
        ],
    )
    def k(cidx_hbm, xidx_hbm, cemb_hbm, xemb_hbm, cb_hbm, xb_hbm,
          ce_out, xe_out, cb_out, xb_out,
          cidx_v, xidx_v, crow_v, xrow_v, ce_v, xe_v,
          cbr_v, xbr_v, cbo_v, xbo_v, sem):
        wid = lax.axis_index("s") * _NC + lax.axis_index("c")
        base = wid * _BPW
        for j in range(_NCH):
            pltpu.sync_copy(cidx_hbm.at[pl.ds(base + j * _CHUNK, _CHUNK)],
                            cidx_v.at[j])
            pltpu.sync_copy(xidx_hbm.at[pl.ds(base + j * _CHUNK, _CHUNK)],
                            xidx_v.at[j])
        for j in range(_NCH):
            for g in range(_NG):
                sl = pl.ds(g * _L, _L)
                cv = cidx_v[j, sl]
                xv = xidx_v[j, sl]
                crow_v[j, sl] = jnp.right_shift(cv, 4)
                xrow_v[j, sl] = jnp.right_shift(xv, 4)
        copies = []
        for j in range(_NCH):
            copies.append(pltpu.async_copy(cemb_hbm.at[cidx_v.at[j]],
                                           ce_v.at[j], sem))
            copies.append(pltpu.async_copy(xemb_hbm.at[xidx_v.at[j]],
                                           xe_v.at[j], sem))
            copies.append(pltpu.async_copy(cb_hbm.at[crow_v.at[j]],
                                           cbr_v.at[j], sem))
            copies.append(pltpu.async_copy(xb_hbm.at[xrow_v.at[j]],
                                           xbr_v.at[j], sem))
        for c in copies:
            c.wait()
        for j in range(_NCH):
            for g in range(_NG):
                sl = pl.ds(g * _L, _L)
                rowpos = lax.iota(jnp.int32, _L) + g * _L
                ccol = jnp.bitwise_and(cidx_v[j, sl], 15)
                xcol = jnp.bitwise_and(xidx_v[j, sl], 15)
                cbo_v[j, sl] = plsc.load_gather(cbr_v.at[j], [rowpos, ccol])
                xbo_v[j, sl] = plsc.load_gather(xbr_v.at[j], [rowpos, xcol])
        for j in range(_NCH):
            dst = pl.ds(base + j * _CHUNK, _CHUNK)
            pltpu.sync_copy(ce_v.at[j], ce_out.at[dst])
            pltpu.sync_copy(xe_v.at[j], xe_out.at[dst])
            pltpu.sync_copy(cbo_v.at[j], cb_out.at[dst])
            pltpu.sync_copy(xbo_v.at[j], xb_out.at[dst])

    return k(center_idx, context_idx, center_embed, context_embed,
             center_bias16, context_bias16)


def kernel(center_idx, context_idx, center_embed, context_embed,
           center_bias, context_bias):
    ce, xe, cb, xb = _glove_gather(
        center_idx.astype(jnp.int32), context_idx.astype(jnp.int32),
        center_embed, context_embed,
        center_bias.reshape(VOCAB // 16, 16),
        context_bias.reshape(VOCAB // 16, 16))
    return (ce, xe, cb, xb)


# speedup vs baseline: 1.0957x; 1.0957x over previous
import functools

import jax
import jax.numpy as jnp
from jax import lax
from jax.experimental import pallas as pl
from jax.experimental.pallas import tpu as pltpu
from jax.experimental.pallas import tpu_sc as plsc

VOCAB = 1000000
EMBED_DIM = 64
BATCH = 16384

_VB = 7812
_VC = _VB * 128
_FLAT = _VB * 1024

_info = plsc.get_sparse_core_info()
_NC, _NS, _L = _info.num_cores, _info.num_subcores, _info.num_lanes
_NW = _NC * _NS
_BPW = BATCH // _NW
_CHUNK = 128
_NCH = _BPW // _CHUNK
_NG = _CHUNK // _L


def _slabs(table):
    x = table[:_VC].T.reshape(8, 8, _VC)
    return [x[fb].reshape(8, _VB, 128).transpose(1, 0, 2).reshape(_FLAT)
            for fb in range(8)]


def _glove_gather(cidx, xidx, cslabs, xslabs, cbias1d, xbias1d):
    mesh = plsc.VectorSubcoreMesh(core_axis_name="c", subcore_axis_name="s")

    @functools.partial(
        pl.kernel,
        mesh=mesh,
        compiler_params=pltpu.CompilerParams(use_tc_tiling_on_sc=True,
                                             needs_layout_passes=False),
        out_type=[
            jax.ShapeDtypeStruct((EMBED_DIM, BATCH), jnp.float32),
            jax.ShapeDtypeStruct((EMBED_DIM, BATCH), jnp.float32),
            jax.ShapeDtypeStruct((BATCH,), jnp.float32),
            jax.ShapeDtypeStruct((BATCH,), jnp.float32),
        ],
        scratch_types=[
            pltpu.VMEM((_NCH, _CHUNK), jnp.int32),
            pltpu.VMEM((_NCH, _CHUNK), jnp.int32),
            pltpu.VMEM((_NCH, 8, _CHUNK), jnp.int32),
            pltpu.VMEM((_NCH, 8, _CHUNK), jnp.int32),
            pltpu.VMEM((_NCH, 8, 8, _CHUNK), jnp.float32),
            pltpu.VMEM((_NCH, 8, 8, _CHUNK), jnp.float32),
            pltpu.VMEM((_NCH, _CHUNK), jnp.float32),
            pltpu.VMEM((_NCH, _CHUNK), jnp.float32),
            pltpu.SemaphoreType.DMA,
        ],
    )
    def k(cidx_hbm, xidx_hbm,
          cs0, cs1, cs2, cs3, cs4, cs5, cs6, cs7,
          xs0, xs1, xs2, xs3, xs4, xs5, xs6, xs7,
          cb_hbm, xb_hbm,
          ceT_out, xeT_out, cb_out, xb_out,
          cidx_v, xidx_v, coff_v, xoff_v, cg_v, xg_v, cbo_v, xbo_v, sem):
        csl = (cs0, cs1, cs2, cs3, cs4, cs5, cs6, cs7)
        xsl = (xs0, xs1, xs2, xs3, xs4, xs5, xs6, xs7)
        wid = lax.axis_index("s") * _NC + lax.axis_index("c")
        base = wid * _BPW

        idx_cps = []
        for j in range(_NCH):
            src = pl.ds(base + j * _CHUNK, _CHUNK)
            idx_cps.append(pltpu.async_copy(cidx_hbm.at[src], cidx_v.at[j],
                                            sem))
            idx_cps.append(pltpu.async_copy(xidx_hbm.at[src], xidx_v.at[j],
                                            sem))
        for c in idx_cps:
            c.wait()

        for j in range(_NCH):
            for g in range(_NG):
                sl = pl.ds(g * _L, _L)
                for iv_ref, off_ref in ((cidx_v, coff_v), (xidx_v, xoff_v)):
                    iv = jnp.minimum(iv_ref[j, sl], _VC - 1)
                    fbase = jnp.right_shift(iv, 7) * 1024 + \
                        jnp.bitwise_and(iv, 127)
                    for fr in range(8):
                        off_ref[j, fr, sl] = fbase + fr * 128

        bias_cps = []
        for j in range(_NCH):
            bias_cps.append(pltpu.async_copy(cb_hbm.at[cidx_v.at[j]],
                                             cbo_v.at[j], sem))
            bias_cps.append(pltpu.async_copy(xb_hbm.at[xidx_v.at[j]],
                                             xbo_v.at[j], sem))

        for fb in range(8):
            for slab, off_ref, g_ref in ((csl[fb], coff_v, cg_v),
                                         (xsl[fb], xoff_v, xg_v)):
                def fire(j, _, slab=slab, off_ref=off_ref, g_ref=g_ref,
                         fb=fb):
                    cps = [pltpu.async_copy(slab.at[off_ref.at[j, fr]],
                                            g_ref.at[j, fb, fr], sem)
                           for fr in range(8)]
                    for c in cps:
                        c.wait()
                    return _
                lax.fori_loop(0, _NCH, fire, 0)

        for c in bias_cps:
            c.wait()

        for fb in range(8):
            def flush(j, _, fb=fb):
                off = pl.multiple_of(base + j * _CHUNK, _CHUNK)
                dst = (pl.ds(fb * 8, 8), pl.ds(off, _CHUNK))
                c1 = pltpu.async_copy(cg_v.at[j, fb], ceT_out.at[dst], sem)
                c2 = pltpu.async_copy(xg_v.at[j, fb], xeT_out.at[dst], sem)
                c1.wait()
                c2.wait()
                return _
            lax.fori_loop(0, _NCH, flush, 0)
        out_cps = []
        for j in range(_NCH):
            dst = pl.ds(base + j * _CHUNK, _CHUNK)
            out_cps.append(pltpu.async_copy(cbo_v.at[j], cb_out.at[dst],
                                            sem))
            out_cps.append(pltpu.async_copy(xbo_v.at[j], xb_out.at[dst],
                                            sem))
        for c in out_cps:
            c.wait()

    return k(cidx, xidx, *cslabs, *xslabs, cbias1d, xbias1d)


def kernel(center_idx, context_idx, center_embed, context_embed,
           center_bias, context_bias):
    cidx = center_idx.astype(jnp.int32)
    xidx = context_idx.astype(jnp.int32)
    ceT, xeT, cb, xb = _glove_gather(
        cidx, xidx, _slabs(center_embed), _slabs(context_embed),
        center_bias.reshape(VOCAB), context_bias.reshape(VOCAB))
    ce, xe = ceT.T, xeT.T
    for idx, tab, main in ((cidx, center_embed, ce),
                           (xidx, context_embed, xe)):
        tail = jnp.take(tab[_VC:], jnp.clip(idx - _VC, 0, VOCAB - _VC - 1),
                        axis=0)
        patched = jnp.where((idx >= _VC)[:, None], tail, main)
        if tab is center_embed:
            ce = patched
        else:
            xe = patched
    return (ce, xe, cb, xb)
